# baseline (speedup 1.0000x reference)
import functools

import jax
import jax.numpy as jnp
from jax import lax
from jax.experimental import pallas as pl
from jax.experimental.pallas import tpu as pltpu
from jax.experimental.pallas import tpu_sc as plsc

N = 10000
E = 640000
F = 42
FP = 48
H = 20
CHUNK = 128
NC = 2
NS = 16
NW = NC * NS
NCHUNKS = E // CHUNK
CPW = -(-NCHUNKS // NW)
ROWS_PER_TILE = 624


def _bcast_lane(v16, l):
    idx = jnp.full((16, 1), l, jnp.int32)
    dn = lax.GatherDimensionNumbers(
        offset_dims=(), collapsed_slice_dims=(0,), start_index_map=(0,))
    return lax.gather(v16, idx, dn, (1,),
                      mode=lax.GatherScatterMode.PROMISE_IN_BOUNDS)


def _edge_stage(x_aug, src, dst, attr):
    mesh = plsc.VectorSubcoreMesh(core_axis_name="c", subcore_axis_name="s")

    @functools.partial(
        pl.kernel,
        out_type=jax.ShapeDtypeStruct((NC, N, FP), jnp.float32),
        mesh=mesh,
        compiler_params=pltpu.CompilerParams(use_tc_tiling_on_sc=False),
        scratch_types=[
            pltpu.VMEM((CHUNK,), jnp.int32),
            pltpu.VMEM((CHUNK,), jnp.int32),
            pltpu.VMEM((CHUNK,), jnp.float32),
            pltpu.VMEM((CHUNK,), jnp.float32),
            pltpu.VMEM((CHUNK, FP), jnp.float32),
            pltpu.VMEM((CHUNK, FP), jnp.float32),
            pltpu.VMEM_SHARED((N, FP), jnp.float32),
            pltpu.SemaphoreType.DMA,
        ],
    )
    def edge_kernel(x_hbm, src_hbm, dst_hbm, attr_hbm, out_hbm,
                    src_v, dst_v, attr_v, gate_v, rows_v, zero_v, acc_sh, sem):
        cid = lax.axis_index("c")
        sid = lax.axis_index("s")
        wid = sid * NC + cid

        def _zrow(j, _):
            for c in range(FP // 16):
                zero_v[j, pl.ds(c * 16, 16)] = jnp.zeros((16,), jnp.float32)
            return 0
        lax.fori_loop(0, CHUNK, _zrow, 0)
        base_row = sid * ROWS_PER_TILE
        for r in range(0, ROWS_PER_TILE, CHUNK):
            nrows = min(CHUNK, ROWS_PER_TILE - r)
            pltpu.sync_copy(zero_v.at[pl.ds(0, nrows)],
                            acc_sh.at[pl.ds(base_row + r, nrows)])

        @pl.when(sid == NS - 1)
        def _():
            pltpu.sync_copy(zero_v.at[pl.ds(0, N - NS * ROWS_PER_TILE)],
                            acc_sh.at[pl.ds(NS * ROWS_PER_TILE,
                                            N - NS * ROWS_PER_TILE)])
        plsc.subcore_barrier()

        def _chunk(i, _):
            g = i * NW + wid

            @pl.when(g < NCHUNKS)
            def _():
                ebase = g * CHUNK
                pltpu.sync_copy(src_hbm.at[pl.ds(ebase, CHUNK)], src_v)
                pltpu.sync_copy(dst_hbm.at[pl.ds(ebase, CHUNK)], dst_v)
                pltpu.sync_copy(attr_hbm.at[pl.ds(ebase, CHUNK)], attr_v)
                for j in range(CHUNK // 16):
                    a = attr_v[pl.ds(j * 16, 16)]
                    gate_v[pl.ds(j * 16, 16)] = 1.0 / (1.0 + jnp.exp(a))
                pltpu.async_copy(x_hbm.at[src_v], rows_v, sem).wait()

                def _scale(jg, _):
                    g16 = gate_v[pl.ds(jg * 16, 16)]
                    for l in range(16):
                        j = jg * 16 + l
                        gj = _bcast_lane(g16, l)
                        for c in range(FP // 16):
                            rv = rows_v[j, pl.ds(c * 16, 16)]
                            rows_v[j, pl.ds(c * 16, 16)] = rv * gj
                    return 0
                lax.fori_loop(0, CHUNK // 16, _scale, 0)

                pltpu.sync_copy(rows_v, acc_sh.at[dst_v], add=True)
            return 0

        lax.fori_loop(0, CPW, _chunk, 0)
        plsc.subcore_barrier()

        pltpu.sync_copy(acc_sh.at[pl.ds(base_row, ROWS_PER_TILE)],
                        out_hbm.at[cid, pl.ds(base_row, ROWS_PER_TILE)])

        @pl.when(sid == NS - 1)
        def _():
            pltpu.sync_copy(
                acc_sh.at[pl.ds(NS * ROWS_PER_TILE, N - NS * ROWS_PER_TILE)],
                out_hbm.at[cid, pl.ds(NS * ROWS_PER_TILE,
                                      N - NS * ROWS_PER_TILE)])

    return edge_kernel(x_aug, src, dst, attr)


RB = 1000
GRID = N // RB


def _dense_body(x_ref, sp_ref, at_ref, bt_ref, blin_ref,
                wih_ref, whh_ref, bih_ref, bhh_ref,
                wif_ref, bf_ref, wib_ref, bb_ref,
                gf_ref, gb_ref):
    s_aug = sp_ref[0] + sp_ref[1]
    s = s_aug[:, :F]
    gsum = s_aug[:, F:F + 1]
    x = x_ref[...]
    xa = jnp.dot(x, at_ref[...], preferred_element_type=jnp.float32)
    agg = gsum * (xa + blin_ref[...]) + jnp.dot(
        s, bt_ref[...], preferred_element_type=jnp.float32)
    gi = jnp.dot(agg, wih_ref[...],
                 preferred_element_type=jnp.float32) + bih_ref[...]
    gh = jnp.dot(agg, whh_ref[...],
                 preferred_element_type=jnp.float32) + bhh_ref[...]
    r = jax.nn.sigmoid(gi[:, :F] + gh[:, :F])
    z = jax.nn.sigmoid(gi[:, F:2 * F] + gh[:, F:2 * F])
    n = jnp.tanh(gi[:, 2 * F:] + r * gh[:, 2 * F:])
    h = (1.0 - z) * n + z * agg
    gf = jnp.dot(h, wif_ref[...],
                 preferred_element_type=jnp.float32) + bf_ref[...]
    gb = jnp.dot(h, wib_ref[...],
                 preferred_element_type=jnp.float32) + bb_ref[...]
    for k in range(4):
        gf_ref[k] = gf[:, k * H:(k + 1) * H]
        gb_ref[k] = gb[:, k * H:(k + 1) * H]


def _dense_stage(x, sp, A_T, B_T, b_lin, WihT, WhhT, b_ih, b_hh,
                 WifT, bf, WibT, bb):
    full = lambda shape: pl.BlockSpec(shape, lambda g: (0,) * len(shape))
    return pl.pallas_call(
        _dense_body,
        grid=(GRID,),
        in_specs=[
            pl.BlockSpec((RB, F), lambda g: (g, 0)),
            pl.BlockSpec((NC, RB, FP), lambda g: (0, g, 0)),
            full((F, F)), full((F, F)), full((1, F)),
            full((F, 3 * F)), full((F, 3 * F)), full((1, 3 * F)),
            full((1, 3 * F)),
            full((F, 4 * H)), full((1, 4 * H)),
            full((F, 4 * H)), full((1, 4 * H)),
        ],
        out_specs=[
            pl.BlockSpec((4, RB, H), lambda g: (0, g, 0)),
            pl.BlockSpec((4, RB, H), lambda g: (0, g, 0)),
        ],
        out_shape=[
            jax.ShapeDtypeStruct((4, N, H), jnp.float32),
            jax.ShapeDtypeStruct((4, N, H), jnp.float32),
        ],
    )(x, sp, A_T, B_T, b_lin, WihT, WhhT, b_ih, b_hh, WifT, bf, WibT, bb)


SB = 16


def _scan_body(gf_ref, gb_ref, whf_ref, whb_ref, hf_ref, hb_ref):
    wf = [whf_ref[:, k * H:(k + 1) * H].astype(jnp.bfloat16) for k in range(4)]
    wb = [whb_ref[:, k * H:(k + 1) * H].astype(jnp.bfloat16) for k in range(4)]

    sig = lambda v: 0.5 * jnp.tanh(0.5 * v) + 0.5

    def blk(b, carry):
        hf, cf, hb, cb = carry
        basef = pl.multiple_of(b * SB, SB)
        baseb = pl.multiple_of(N - SB - b * SB, SB)
        GF = [gf_ref[k, pl.ds(basef, SB), :] for k in range(4)]
        GB = [gb_ref[k, pl.ds(baseb, SB), :] for k in range(4)]
        hrowsf, hrowsb = [], []
        for j in range(SB):
            jb = SB - 1 - j
            hfl = hf.astype(jnp.bfloat16)
            hbl = hb.astype(jnp.bfloat16)
            pf = [GF[k][j:j + 1, :] + jnp.dot(
                hfl, wf[k], preferred_element_type=jnp.float32)
                for k in range(4)]
            pb = [GB[k][jb:jb + 1, :] + jnp.dot(
                hbl, wb[k], preferred_element_type=jnp.float32)
                for k in range(4)]
            cf = sig(pf[1]) * cf + sig(pf[0]) * jnp.tanh(pf[2])
            cb = sig(pb[1]) * cb + sig(pb[0]) * jnp.tanh(pb[2])
            hf = sig(pf[3]) * jnp.tanh(cf)
            hb = sig(pb[3]) * jnp.tanh(cb)
            hrowsf.append(hf)
            hrowsb.append(hb)
        HF = jnp.concatenate(hrowsf, axis=0)
        HB = jnp.concatenate(list(reversed(hrowsb)), axis=0)
        hf_ref[pl.ds(basef, SB), :] = HF
        hb_ref[pl.ds(baseb, SB), :] = HB
        return hf, cf, hb, cb

    z = jnp.zeros((1, H), jnp.float32)
    lax.fori_loop(0, N // SB, blk, (z, z, z, z))


def _scan_stage(Gf4, Gb4, WhfT, WhbT):
    return pl.pallas_call(
        _scan_body,
        out_shape=[
            jax.ShapeDtypeStruct((N, H), jnp.float32),
            jax.ShapeDtypeStruct((N, H), jnp.float32),
        ],
    )(Gf4, Gb4, WhfT, WhbT)


def _proj_body(hf_ref, hb_ref, wcf_ref, wcb_ref, bc_ref, out_ref):
    out_ref[...] = (
        jnp.dot(hf_ref[...], wcf_ref[...], preferred_element_type=jnp.float32)
        + jnp.dot(hb_ref[...], wcb_ref[...],
                  preferred_element_type=jnp.float32)
        + bc_ref[...])


def _proj_stage(Hf, Hb, WcfT, WcbT, b_c):
    full = lambda shape: pl.BlockSpec(shape, lambda g: (0,) * len(shape))
    return pl.pallas_call(
        _proj_body,
        grid=(GRID,),
        in_specs=[
            pl.BlockSpec((RB, H), lambda g: (g, 0)),
            pl.BlockSpec((RB, H), lambda g: (g, 0)),
            full((H, 1)), full((H, 1)), full((1, 1)),
        ],
        out_specs=pl.BlockSpec((RB, 1), lambda g: (g, 0)),
        out_shape=jax.ShapeDtypeStruct((N, 1), jnp.float32),
    )(Hf, Hb, WcfT, WcbT, b_c)


def kernel(x, edge_index, edge_attr, W_lin, b_lin, gru_W_ih, gru_W_hh,
           gru_b_ih, gru_b_hh, W_ih_f, W_hh_f, b_ih_f, b_hh_f,
           W_ih_b, W_hh_b, b_ih_b, b_hh_b, W_c, b_c):
    x_aug = jnp.concatenate(
        [x, jnp.ones((N, 1), jnp.float32), jnp.zeros((N, FP - F - 1),
                                                     jnp.float32)], axis=1)
    src = edge_index[0]
    dst = edge_index[1]
    attr = edge_attr[:, 0]

    sp = _edge_stage(x_aug, src, dst, attr)

    Gf, Gb = _dense_stage(
        x, sp,
        W_lin[:, :F].T, W_lin[:, F:].T, b_lin.reshape(1, F),
        gru_W_ih.T, gru_W_hh.T,
        gru_b_ih.reshape(1, 3 * F), gru_b_hh.reshape(1, 3 * F),
        W_ih_f.T, (b_ih_f + b_hh_f).reshape(1, 4 * H),
        W_ih_b.T, (b_ih_b + b_hh_b).reshape(1, 4 * H),
    )

    Hf, Hb = _scan_stage(Gf, Gb, W_hh_f.T, W_hh_b.T)

    return _proj_stage(Hf, Hb, W_c[:, :H].T, W_c[:, H:].T,
                       b_c.reshape(1, 1))


# speedup vs baseline: 1.0494x; 1.0494x over previous
import functools

import jax
import jax.numpy as jnp
from jax import lax
from jax.experimental import pallas as pl
from jax.experimental.pallas import tpu as pltpu
from jax.experimental.pallas import tpu_sc as plsc

N = 10000
E = 640000
F = 42
FP = 48
H = 20
CHUNK = 128
NC = 2
NS = 16
NW = NC * NS
NCHUNKS = E // CHUNK
CPW = -(-NCHUNKS // NW)
ROWS_PER_TILE = 624


def _bcast_lane(v16, l):
    idx = jnp.full((16, 1), l, jnp.int32)
    dn = lax.GatherDimensionNumbers(
        offset_dims=(), collapsed_slice_dims=(0,), start_index_map=(0,))
    return lax.gather(v16, idx, dn, (1,),
                      mode=lax.GatherScatterMode.PROMISE_IN_BOUNDS)


def _edge_stage(x_aug, src, dst, attr):
    mesh = plsc.VectorSubcoreMesh(core_axis_name="c", subcore_axis_name="s")

    @functools.partial(
        pl.kernel,
        out_type=jax.ShapeDtypeStruct((NC, N, FP), jnp.float32),
        mesh=mesh,
        compiler_params=pltpu.CompilerParams(use_tc_tiling_on_sc=False),
        scratch_types=[
            pltpu.VMEM((CHUNK,), jnp.int32),
            pltpu.VMEM((CHUNK,), jnp.int32),
            pltpu.VMEM((CHUNK,), jnp.float32),
            pltpu.VMEM((CHUNK,), jnp.float32),
            pltpu.VMEM((CHUNK, FP), jnp.float32),
            pltpu.VMEM((CHUNK,), jnp.int32),
            pltpu.VMEM((CHUNK,), jnp.int32),
            pltpu.VMEM((CHUNK,), jnp.float32),
            pltpu.VMEM((CHUNK,), jnp.float32),
            pltpu.VMEM((CHUNK, FP), jnp.float32),
            pltpu.VMEM((CHUNK, FP), jnp.float32),
            pltpu.VMEM_SHARED((N, FP), jnp.float32),
            pltpu.SemaphoreType.DMA,
            pltpu.SemaphoreType.DMA,
        ],
    )
    def edge_kernel(x_hbm, src_hbm, dst_hbm, attr_hbm, out_hbm,
                    src_a, dst_a, attr_a, gate_a, rows_a,
                    src_b, dst_b, attr_b, gate_b, rows_b,
                    zero_v, acc_sh, sem_a, sem_b):
        cid = lax.axis_index("c")
        sid = lax.axis_index("s")
        wid = sid * NC + cid

        def _zrow(j, _):
            for c in range(FP // 16):
                zero_v[j, pl.ds(c * 16, 16)] = jnp.zeros((16,), jnp.float32)
            return 0
        lax.fori_loop(0, CHUNK, _zrow, 0)
        base_row = sid * ROWS_PER_TILE
        for r in range(0, ROWS_PER_TILE, CHUNK):
            nrows = min(CHUNK, ROWS_PER_TILE - r)
            pltpu.sync_copy(zero_v.at[pl.ds(0, nrows)],
                            acc_sh.at[pl.ds(base_row + r, nrows)])

        @pl.when(sid == NS - 1)
        def _():
            pltpu.sync_copy(zero_v.at[pl.ds(0, N - NS * ROWS_PER_TILE)],
                            acc_sh.at[pl.ds(NS * ROWS_PER_TILE,
                                            N - NS * ROWS_PER_TILE)])
        plsc.subcore_barrier()

        def _start(g, src_v, dst_v, attr_v, gate_v, rows_v, sem):
            ebase = g * CHUNK
            pltpu.sync_copy(src_hbm.at[pl.ds(ebase, CHUNK)], src_v)
            pltpu.sync_copy(dst_hbm.at[pl.ds(ebase, CHUNK)], dst_v)
            pltpu.sync_copy(attr_hbm.at[pl.ds(ebase, CHUNK)], attr_v)
            pltpu.async_copy(x_hbm.at[src_v], rows_v, sem)
            for j in range(CHUNK // 16):
                a = attr_v[pl.ds(j * 16, 16)]
                gate_v[pl.ds(j * 16, 16)] = 1.0 / (1.0 + jnp.exp(a))

        def _consume(gate_v, rows_v, dst_v, sem):
            pltpu.make_async_copy(x_hbm.at[pl.ds(0, CHUNK)], rows_v,
                                  sem).wait()

            def _scale(jg, _):
                g16 = gate_v[pl.ds(jg * 16, 16)]
                for l in range(16):
                    j = jg * 16 + l
                    gj = _bcast_lane(g16, l)
                    for c in range(FP // 16):
                        rv = rows_v[j, pl.ds(c * 16, 16)]
                        rows_v[j, pl.ds(c * 16, 16)] = rv * gj
                return 0
            lax.fori_loop(0, CHUNK // 16, _scale, 0)

            pltpu.sync_copy(rows_v, acc_sh.at[dst_v], add=True)

        _start(wid, src_a, dst_a, attr_a, gate_a, rows_a, sem_a)

        def _pair(ip, _):
            ga = (2 * ip) * NW + wid
            gb = ga + NW
            ga2 = ga + 2 * NW

            @pl.when(gb < NCHUNKS)
            def _():
                _start(gb, src_b, dst_b, attr_b, gate_b, rows_b, sem_b)

            @pl.when(ga < NCHUNKS)
            def _():
                _consume(gate_a, rows_a, dst_a, sem_a)

            @pl.when(ga2 < NCHUNKS)
            def _():
                _start(ga2, src_a, dst_a, attr_a, gate_a, rows_a, sem_a)

            @pl.when(gb < NCHUNKS)
            def _():
                _consume(gate_b, rows_b, dst_b, sem_b)
            return 0

        lax.fori_loop(0, -(-CPW // 2), _pair, 0)
        plsc.subcore_barrier()

        pltpu.sync_copy(acc_sh.at[pl.ds(base_row, ROWS_PER_TILE)],
                        out_hbm.at[cid, pl.ds(base_row, ROWS_PER_TILE)])

        @pl.when(sid == NS - 1)
        def _():
            pltpu.sync_copy(
                acc_sh.at[pl.ds(NS * ROWS_PER_TILE, N - NS * ROWS_PER_TILE)],
                out_hbm.at[cid, pl.ds(NS * ROWS_PER_TILE,
                                      N - NS * ROWS_PER_TILE)])

    return edge_kernel(x_aug, src, dst, attr)


RB = 1000
GRID = N // RB


def _dense_body(x_ref, sp_ref, at_ref, bt_ref, blin_ref,
                wih_ref, whh_ref, bih_ref, bhh_ref,
                wif_ref, bf_ref, wib_ref, bb_ref,
                gf_ref, gb_ref):
    s_aug = sp_ref[0] + sp_ref[1]
    s = s_aug[:, :F]
    gsum = s_aug[:, F:F + 1]
    x = x_ref[...]
    xa = jnp.dot(x, at_ref[...], preferred_element_type=jnp.float32)
    agg = gsum * (xa + blin_ref[...]) + jnp.dot(
        s, bt_ref[...], preferred_element_type=jnp.float32)
    gi = jnp.dot(agg, wih_ref[...],
                 preferred_element_type=jnp.float32) + bih_ref[...]
    gh = jnp.dot(agg, whh_ref[...],
                 preferred_element_type=jnp.float32) + bhh_ref[...]
    r = jax.nn.sigmoid(gi[:, :F] + gh[:, :F])
    z = jax.nn.sigmoid(gi[:, F:2 * F] + gh[:, F:2 * F])
    n = jnp.tanh(gi[:, 2 * F:] + r * gh[:, 2 * F:])
    h = (1.0 - z) * n + z * agg
    gf = jnp.dot(h, wif_ref[...],
                 preferred_element_type=jnp.float32) + bf_ref[...]
    gb = jnp.dot(h, wib_ref[...],
                 preferred_element_type=jnp.float32) + bb_ref[...]
    for k in range(4):
        gf_ref[k] = gf[:, k * H:(k + 1) * H]
        gb_ref[k] = gb[:, k * H:(k + 1) * H]


def _dense_stage(x, sp, A_T, B_T, b_lin, WihT, WhhT, b_ih, b_hh,
                 WifT, bf, WibT, bb):
    full = lambda shape: pl.BlockSpec(shape, lambda g: (0,) * len(shape))
    return pl.pallas_call(
        _dense_body,
        grid=(GRID,),
        in_specs=[
            pl.BlockSpec((RB, F), lambda g: (g, 0)),
            pl.BlockSpec((NC, RB, FP), lambda g: (0, g, 0)),
            full((F, F)), full((F, F)), full((1, F)),
            full((F, 3 * F)), full((F, 3 * F)), full((1, 3 * F)),
            full((1, 3 * F)),
            full((F, 4 * H)), full((1, 4 * H)),
            full((F, 4 * H)), full((1, 4 * H)),
        ],
        out_specs=[
            pl.BlockSpec((4, RB, H), lambda g: (0, g, 0)),
            pl.BlockSpec((4, RB, H), lambda g: (0, g, 0)),
        ],
        out_shape=[
            jax.ShapeDtypeStruct((4, N, H), jnp.float32),
            jax.ShapeDtypeStruct((4, N, H), jnp.float32),
        ],
    )(x, sp, A_T, B_T, b_lin, WihT, WhhT, b_ih, b_hh, WifT, bf, WibT, bb)


SB = 16


def _scan_body(gf_ref, gb_ref, whf_ref, whb_ref, hf_ref, hb_ref):
    wf = [whf_ref[:, k * H:(k + 1) * H].astype(jnp.bfloat16) for k in range(4)]
    wb = [whb_ref[:, k * H:(k + 1) * H].astype(jnp.bfloat16) for k in range(4)]

    sig = jax.nn.sigmoid

    def blk(b, carry):
        hf, cf, hb, cb = carry
        basef = pl.multiple_of(b * SB, SB)
        baseb = pl.multiple_of(N - SB - b * SB, SB)
        GF = [gf_ref[k, pl.ds(basef, SB), :] for k in range(4)]
        GB = [gb_ref[k, pl.ds(baseb, SB), :] for k in range(4)]
        hrowsf, hrowsb = [], []
        for j in range(SB):
            jb = SB - 1 - j
            hfl = hf.astype(jnp.bfloat16)
            hbl = hb.astype(jnp.bfloat16)
            pf = [GF[k][j:j + 1, :] + jnp.dot(
                hfl, wf[k], preferred_element_type=jnp.float32)
                for k in range(4)]
            pb = [GB[k][jb:jb + 1, :] + jnp.dot(
                hbl, wb[k], preferred_element_type=jnp.float32)
                for k in range(4)]
            cf = sig(pf[1]) * cf + sig(pf[0]) * jnp.tanh(pf[2])
            cb = sig(pb[1]) * cb + sig(pb[0]) * jnp.tanh(pb[2])
            hf = sig(pf[3]) * jnp.tanh(cf)
            hb = sig(pb[3]) * jnp.tanh(cb)
            hrowsf.append(hf)
            hrowsb.append(hb)
        HF = jnp.concatenate(hrowsf, axis=0)
        HB = jnp.concatenate(list(reversed(hrowsb)), axis=0)
        hf_ref[pl.ds(basef, SB), :] = HF
        hb_ref[pl.ds(baseb, SB), :] = HB
        return hf, cf, hb, cb

    z = jnp.zeros((1, H), jnp.float32)
    lax.fori_loop(0, N // SB, blk, (z, z, z, z))


def _scan_stage(Gf4, Gb4, WhfT, WhbT):
    return pl.pallas_call(
        _scan_body,
        out_shape=[
            jax.ShapeDtypeStruct((N, H), jnp.float32),
            jax.ShapeDtypeStruct((N, H), jnp.float32),
        ],
    )(Gf4, Gb4, WhfT, WhbT)


def _proj_body(hf_ref, hb_ref, wcf_ref, wcb_ref, bc_ref, out_ref):
    out_ref[...] = (
        jnp.dot(hf_ref[...], wcf_ref[...], preferred_element_type=jnp.float32)
        + jnp.dot(hb_ref[...], wcb_ref[...],
                  preferred_element_type=jnp.float32)
        + bc_ref[...])


def _proj_stage(Hf, Hb, WcfT, WcbT, b_c):
    full = lambda shape: pl.BlockSpec(shape, lambda g: (0,) * len(shape))
    return pl.pallas_call(
        _proj_body,
        grid=(GRID,),
        in_specs=[
            pl.BlockSpec((RB, H), lambda g: (g, 0)),
            pl.BlockSpec((RB, H), lambda g: (g, 0)),
            full((H, 1)), full((H, 1)), full((1, 1)),
        ],
        out_specs=pl.BlockSpec((RB, 1), lambda g: (g, 0)),
        out_shape=jax.ShapeDtypeStruct((N, 1), jnp.float32),
    )(Hf, Hb, WcfT, WcbT, b_c)


def kernel(x, edge_index, edge_attr, W_lin, b_lin, gru_W_ih, gru_W_hh,
           gru_b_ih, gru_b_hh, W_ih_f, W_hh_f, b_ih_f, b_hh_f,
           W_ih_b, W_hh_b, b_ih_b, b_hh_b, W_c, b_c):
    x_aug = jnp.concatenate(
        [x, jnp.ones((N, 1), jnp.float32), jnp.zeros((N, FP - F - 1),
                                                     jnp.float32)], axis=1)
    src = edge_index[0]
    dst = edge_index[1]
    attr = edge_attr[:, 0]

    sp = _edge_stage(x_aug, src, dst, attr)

    Gf, Gb = _dense_stage(
        x, sp,
        W_lin[:, :F].T, W_lin[:, F:].T, b_lin.reshape(1, F),
        gru_W_ih.T, gru_W_hh.T,
        gru_b_ih.reshape(1, 3 * F), gru_b_hh.reshape(1, 3 * F),
        W_ih_f.T, (b_ih_f + b_hh_f).reshape(1, 4 * H),
        W_ih_b.T, (b_ih_b + b_hh_b).reshape(1, 4 * H),
    )

    Hf, Hb = _scan_stage(Gf, Gb, W_hh_f.T, W_hh_b.T)

    return _proj_stage(Hf, Hb, W_c[:, :H].T, W_c[:, H:].T,
                       b_c.reshape(1, 1))


# speedup vs baseline: 1.0726x; 1.0221x over previous
import functools

import jax
import jax.numpy as jnp
from jax import lax
from jax.experimental import pallas as pl
from jax.experimental.pallas import tpu as pltpu
from jax.experimental.pallas import tpu_sc as plsc

N = 10000
E = 640000
F = 42
FP = 48
H = 20
CHUNK = 128
NC = 2
NS = 16
NW = NC * NS
NCHUNKS = E // CHUNK
CPW = -(-NCHUNKS // NW)
ROWS_PER_TILE = 624


def _bcast_lane(v16, l):
    idx = jnp.full((16, 1), l, jnp.int32)
    dn = lax.GatherDimensionNumbers(
        offset_dims=(), collapsed_slice_dims=(0,), start_index_map=(0,))
    return lax.gather(v16, idx, dn, (1,),
                      mode=lax.GatherScatterMode.PROMISE_IN_BOUNDS)


def _edge_stage(x_aug, src, dst, attr):
    mesh = plsc.VectorSubcoreMesh(core_axis_name="c", subcore_axis_name="s")

    @functools.partial(
        pl.kernel,
        out_type=jax.ShapeDtypeStruct((NC, N, FP), jnp.float32),
        mesh=mesh,
        compiler_params=pltpu.CompilerParams(use_tc_tiling_on_sc=False),
        scratch_types=[
            pltpu.VMEM((CHUNK,), jnp.int32),
            pltpu.VMEM((CHUNK,), jnp.int32),
            pltpu.VMEM((CHUNK,), jnp.float32),
            pltpu.VMEM((CHUNK,), jnp.float32),
            pltpu.VMEM((CHUNK, FP), jnp.float32),
            pltpu.VMEM((CHUNK,), jnp.int32),
            pltpu.VMEM((CHUNK,), jnp.int32),
            pltpu.VMEM((CHUNK,), jnp.float32),
            pltpu.VMEM((CHUNK,), jnp.float32),
            pltpu.VMEM((CHUNK, FP), jnp.float32),
            pltpu.VMEM((CHUNK, FP), jnp.float32),
            pltpu.VMEM_SHARED((N, FP), jnp.float32),
            pltpu.SemaphoreType.DMA,
            pltpu.SemaphoreType.DMA,
            pltpu.SemaphoreType.DMA,
            pltpu.SemaphoreType.DMA,
        ],
    )
    def edge_kernel(x_hbm, src_hbm, dst_hbm, attr_hbm, out_hbm,
                    src_a, dst_a, attr_a, gate_a, rows_a,
                    src_b, dst_b, attr_b, gate_b, rows_b,
                    zero_v, acc_sh, sem_a, sem_b, sem_sa, sem_sb):
        cid = lax.axis_index("c")
        sid = lax.axis_index("s")
        wid = sid * NC + cid

        def _zrow(j, _):
            for c in range(FP // 16):
                zero_v[j, pl.ds(c * 16, 16)] = jnp.zeros((16,), jnp.float32)
            return 0
        lax.fori_loop(0, CHUNK, _zrow, 0)
        base_row = sid * ROWS_PER_TILE
        for r in range(0, ROWS_PER_TILE, CHUNK):
            nrows = min(CHUNK, ROWS_PER_TILE - r)
            pltpu.sync_copy(zero_v.at[pl.ds(0, nrows)],
                            acc_sh.at[pl.ds(base_row + r, nrows)])

        @pl.when(sid == NS - 1)
        def _():
            pltpu.sync_copy(zero_v.at[pl.ds(0, N - NS * ROWS_PER_TILE)],
                            acc_sh.at[pl.ds(NS * ROWS_PER_TILE,
                                            N - NS * ROWS_PER_TILE)])
        plsc.subcore_barrier()

        def _drain(rows_v, sem):
            pltpu.make_async_copy(x_hbm.at[pl.ds(0, CHUNK)], rows_v,
                                  sem).wait()

        def _start(g, src_v, dst_v, attr_v, gate_v, rows_v, sem, sem_s,
                   drain_scatter):
            ebase = g * CHUNK
            pltpu.sync_copy(src_hbm.at[pl.ds(ebase, CHUNK)], src_v)
            pltpu.sync_copy(dst_hbm.at[pl.ds(ebase, CHUNK)], dst_v)
            pltpu.sync_copy(attr_hbm.at[pl.ds(ebase, CHUNK)], attr_v)
            if drain_scatter:
                _drain(rows_v, sem_s)
            pltpu.async_copy(x_hbm.at[src_v], rows_v, sem)
            for j in range(CHUNK // 16):
                a = attr_v[pl.ds(j * 16, 16)]
                gate_v[pl.ds(j * 16, 16)] = 1.0 / (1.0 + jnp.exp(a))

        def _consume(gate_v, rows_v, dst_v, sem, sem_s):
            _drain(rows_v, sem)

            def _scale(jg, _):
                g16 = gate_v[pl.ds(jg * 16, 16)]
                for l in range(16):
                    j = jg * 16 + l
                    gj = _bcast_lane(g16, l)
                    for c in range(FP // 16):
                        rv = rows_v[j, pl.ds(c * 16, 16)]
                        rows_v[j, pl.ds(c * 16, 16)] = rv * gj
                return 0
            lax.fori_loop(0, CHUNK // 16, _scale, 0)

            pltpu.async_copy(rows_v, acc_sh.at[dst_v], sem_s, add=True)

        _start(wid, src_a, dst_a, attr_a, gate_a, rows_a, sem_a, sem_sa,
               False)
        _start(wid + NW, src_b, dst_b, attr_b, gate_b, rows_b, sem_b,
               sem_sb, False)

        def _pair(ip, _):
            ga = (2 * ip) * NW + wid
            gb = ga + NW
            ga2 = ga + 2 * NW
            gb2 = gb + 2 * NW

            @pl.when(ga < NCHUNKS)
            def _():
                _consume(gate_a, rows_a, dst_a, sem_a, sem_sa)

            @pl.when(ga2 < NCHUNKS)
            def _():
                _start(ga2, src_a, dst_a, attr_a, gate_a, rows_a, sem_a,
                       sem_sa, True)

            @pl.when(gb < NCHUNKS)
            def _():
                _consume(gate_b, rows_b, dst_b, sem_b, sem_sb)

            @pl.when(gb2 < NCHUNKS)
            def _():
                _start(gb2, src_b, dst_b, attr_b, gate_b, rows_b, sem_b,
                       sem_sb, True)
            return 0

        lax.fori_loop(0, -(-CPW // 2), _pair, 0)
        _drain(rows_a, sem_sa)
        _drain(rows_b, sem_sb)
        plsc.subcore_barrier()

        pltpu.sync_copy(acc_sh.at[pl.ds(base_row, ROWS_PER_TILE)],
                        out_hbm.at[cid, pl.ds(base_row, ROWS_PER_TILE)])

        @pl.when(sid == NS - 1)
        def _():
            pltpu.sync_copy(
                acc_sh.at[pl.ds(NS * ROWS_PER_TILE, N - NS * ROWS_PER_TILE)],
                out_hbm.at[cid, pl.ds(NS * ROWS_PER_TILE,
                                      N - NS * ROWS_PER_TILE)])

    return edge_kernel(x_aug, src, dst, attr)


RB = 1000
GRID = N // RB


def _dense_body(x_ref, sp_ref, at_ref, bt_ref, blin_ref,
                wih_ref, whh_ref, bih_ref, bhh_ref,
                wif_ref, bf_ref, wib_ref, bb_ref,
                gf_ref, gb_ref):
    s_aug = sp_ref[0] + sp_ref[1]
    s = s_aug[:, :F]
    gsum = s_aug[:, F:F + 1]
    x = x_ref[...]
    xa = jnp.dot(x, at_ref[...], preferred_element_type=jnp.float32)
    agg = gsum * (xa + blin_ref[...]) + jnp.dot(
        s, bt_ref[...], preferred_element_type=jnp.float32)
    gi = jnp.dot(agg, wih_ref[...],
                 preferred_element_type=jnp.float32) + bih_ref[...]
    gh = jnp.dot(agg, whh_ref[...],
                 preferred_element_type=jnp.float32) + bhh_ref[...]
    r = jax.nn.sigmoid(gi[:, :F] + gh[:, :F])
    z = jax.nn.sigmoid(gi[:, F:2 * F] + gh[:, F:2 * F])
    n = jnp.tanh(gi[:, 2 * F:] + r * gh[:, 2 * F:])
    h = (1.0 - z) * n + z * agg
    gf = jnp.dot(h, wif_ref[...],
                 preferred_element_type=jnp.float32) + bf_ref[...]
    gb = jnp.dot(h, wib_ref[...],
                 preferred_element_type=jnp.float32) + bb_ref[...]
    for k in range(4):
        gf_ref[k] = gf[:, k * H:(k + 1) * H]
        gb_ref[k] = gb[:, k * H:(k + 1) * H]


def _dense_stage(x, sp, A_T, B_T, b_lin, WihT, WhhT, b_ih, b_hh,
                 WifT, bf, WibT, bb):
    full = lambda shape: pl.BlockSpec(shape, lambda g: (0,) * len(shape))
    return pl.pallas_call(
        _dense_body,
        grid=(GRID,),
        in_specs=[
            pl.BlockSpec((RB, F), lambda g: (g, 0)),
            pl.BlockSpec((NC, RB, FP), lambda g: (0, g, 0)),
            full((F, F)), full((F, F)), full((1, F)),
            full((F, 3 * F)), full((F, 3 * F)), full((1, 3 * F)),
            full((1, 3 * F)),
            full((F, 4 * H)), full((1, 4 * H)),
            full((F, 4 * H)), full((1, 4 * H)),
        ],
        out_specs=[
            pl.BlockSpec((4, RB, H), lambda g: (0, g, 0)),
            pl.BlockSpec((4, RB, H), lambda g: (0, g, 0)),
        ],
        out_shape=[
            jax.ShapeDtypeStruct((4, N, H), jnp.float32),
            jax.ShapeDtypeStruct((4, N, H), jnp.float32),
        ],
    )(x, sp, A_T, B_T, b_lin, WihT, WhhT, b_ih, b_hh, WifT, bf, WibT, bb)


SB = 16


def _scan_body(gf_ref, gb_ref, whf_ref, whb_ref, hf_ref, hb_ref):
    wf = [whf_ref[:, k * H:(k + 1) * H].astype(jnp.bfloat16) for k in range(4)]
    wb = [whb_ref[:, k * H:(k + 1) * H].astype(jnp.bfloat16) for k in range(4)]

    sig = jax.nn.sigmoid

    def blk(b, carry):
        hf, cf, hb, cb = carry
        basef = pl.multiple_of(b * SB, SB)
        baseb = pl.multiple_of(N - SB - b * SB, SB)
        GF = [gf_ref[k, pl.ds(basef, SB), :] for k in range(4)]
        GB = [gb_ref[k, pl.ds(baseb, SB), :] for k in range(4)]
        hrowsf, hrowsb = [], []
        for j in range(SB):
            jb = SB - 1 - j
            hfl = hf.astype(jnp.bfloat16)
            hbl = hb.astype(jnp.bfloat16)
            pf = [GF[k][j:j + 1, :] + jnp.dot(
                hfl, wf[k], preferred_element_type=jnp.float32)
                for k in range(4)]
            pb = [GB[k][jb:jb + 1, :] + jnp.dot(
                hbl, wb[k], preferred_element_type=jnp.float32)
                for k in range(4)]
            cf = sig(pf[1]) * cf + sig(pf[0]) * jnp.tanh(pf[2])
            cb = sig(pb[1]) * cb + sig(pb[0]) * jnp.tanh(pb[2])
            hf = sig(pf[3]) * jnp.tanh(cf)
            hb = sig(pb[3]) * jnp.tanh(cb)
            hrowsf.append(hf)
            hrowsb.append(hb)
        HF = jnp.concatenate(hrowsf, axis=0)
        HB = jnp.concatenate(list(reversed(hrowsb)), axis=0)
        hf_ref[pl.ds(basef, SB), :] = HF
        hb_ref[pl.ds(baseb, SB), :] = HB
        return hf, cf, hb, cb

    z = jnp.zeros((1, H), jnp.float32)
    lax.fori_loop(0, N // SB, blk, (z, z, z, z))


def _scan_stage(Gf4, Gb4, WhfT, WhbT):
    return pl.pallas_call(
        _scan_body,
        out_shape=[
            jax.ShapeDtypeStruct((N, H), jnp.float32),
            jax.ShapeDtypeStruct((N, H), jnp.float32),
        ],
    )(Gf4, Gb4, WhfT, WhbT)


def _proj_body(hf_ref, hb_ref, wcf_ref, wcb_ref, bc_ref, out_ref):
    out_ref[...] = (
        jnp.dot(hf_ref[...], wcf_ref[...], preferred_element_type=jnp.float32)
        + jnp.dot(hb_ref[...], wcb_ref[...],
                  preferred_element_type=jnp.float32)
        + bc_ref[...])


def _proj_stage(Hf, Hb, WcfT, WcbT, b_c):
    full = lambda shape: pl.BlockSpec(shape, lambda g: (0,) * len(shape))
    return pl.pallas_call(
        _proj_body,
        grid=(GRID,),
        in_specs=[
            pl.BlockSpec((RB, H), lambda g: (g, 0)),
            pl.BlockSpec((RB, H), lambda g: (g, 0)),
            full((H, 1)), full((H, 1)), full((1, 1)),
        ],
        out_specs=pl.BlockSpec((RB, 1), lambda g: (g, 0)),
        out_shape=jax.ShapeDtypeStruct((N, 1), jnp.float32),
    )(Hf, Hb, WcfT, WcbT, b_c)


def kernel(x, edge_index, edge_attr, W_lin, b_lin, gru_W_ih, gru_W_hh,
           gru_b_ih, gru_b_hh, W_ih_f, W_hh_f, b_ih_f, b_hh_f,
           W_ih_b, W_hh_b, b_ih_b, b_hh_b, W_c, b_c):
    x_aug = jnp.concatenate(
        [x, jnp.ones((N, 1), jnp.float32), jnp.zeros((N, FP - F - 1),
                                                     jnp.float32)], axis=1)
    src = edge_index[0]
    dst = edge_index[1]
    attr = edge_attr[:, 0]

    sp = _edge_stage(x_aug, src, dst, attr)

    Gf, Gb = _dense_stage(
        x, sp,
        W_lin[:, :F].T, W_lin[:, F:].T, b_lin.reshape(1, F),
        gru_W_ih.T, gru_W_hh.T,
        gru_b_ih.reshape(1, 3 * F), gru_b_hh.reshape(1, 3 * F),
        W_ih_f.T, (b_ih_f + b_hh_f).reshape(1, 4 * H),
        W_ih_b.T, (b_ih_b + b_hh_b).reshape(1, 4 * H),
    )

    Hf, Hb = _scan_stage(Gf, Gb, W_hh_f.T, W_hh_b.T)

    return _proj_stage(Hf, Hb, W_c[:, :H].T, W_c[:, H:].T,
                       b_c.reshape(1, 1))
